# baseline (speedup 1.0000x reference)
import functools

import jax
import jax.numpy as jnp
from jax.experimental import pallas as pl
from jax.experimental.pallas import tpu as pltpu

_NSPLIT = 4


def _se_gate_kernel(x0_ref, x1_ref, x2_ref, x3_ref,
                    w1t_ref, b1_ref, w2t_ref, b2_ref, g_ref, *, inv_hw):
    parts = [jnp.sum(x[...], axis=-1) for x in (x0_ref, x1_ref, x2_ref, x3_ref)]
    pooled = jnp.concatenate(parts, axis=-1) * inv_hw
    z = jnp.maximum(
        jnp.dot(pooled, w1t_ref[...], preferred_element_type=jnp.float32)
        + b1_ref[...],
        0.0,
    )
    g_ref[...] = jax.nn.sigmoid(
        jnp.dot(z, w2t_ref[...], preferred_element_type=jnp.float32)
        + b2_ref[...]
    )[:, None, :]


def kernel(enc, dec, w1, b1, w2, b2):
    B, C, H, W = enc.shape
    Csq = w1.shape[0]
    HW = H * W
    Cg = C // _NSPLIT

    enc2 = enc.reshape(B, C, HW)
    w1t = jnp.transpose(w1)
    w2t = jnp.transpose(w2)
    b1r = b1.reshape(1, Csq)
    b2r = b2.reshape(1, C)

    body = functools.partial(_se_gate_kernel, inv_hw=1.0 / HW)

    def enc_spec(i):
        return pl.BlockSpec((1, Cg, HW), lambda b, i=i: (b, i, 0))

    g3 = pl.pallas_call(
        body,
        out_shape=jax.ShapeDtypeStruct((B, 1, C), jnp.float32),
        grid=(B,),
        in_specs=[
            enc_spec(0), enc_spec(1), enc_spec(2), enc_spec(3),
            pl.BlockSpec((C, Csq), lambda b: (0, 0)),
            pl.BlockSpec((1, Csq), lambda b: (0, 0)),
            pl.BlockSpec((Csq, C), lambda b: (0, 0)),
            pl.BlockSpec((1, C), lambda b: (0, 0)),
        ],
        out_specs=pl.BlockSpec((1, 1, C), lambda b: (b, 0, 0)),
        compiler_params=pltpu.CompilerParams(
            dimension_semantics=("parallel",),
            vmem_limit_bytes=100 * 1024 * 1024,
        ),
    )(enc2, enc2, enc2, enc2, w1t, b1r, w2t, b2r)

    g = g3.reshape(B, C)
    se = enc * g[:, :, None, None].astype(enc.dtype)
    return jnp.concatenate([dec, se], axis=1)
